# Initial kernel scaffold; baseline (speedup 1.0000x reference)
import jax
import jax.numpy as jnp
from jax.experimental import pallas as pl


def kernel(W):
    raise NotImplementedError("write your pallas kernel here")


# speedup vs baseline: 1.4410x; 1.4410x over previous
import jax
import jax.numpy as jnp
from jax import lax
from jax.experimental import pallas as pl
from jax.experimental.pallas import tpu as pltpu
from jax.experimental.pallas import tpu_sc as plsc

NUM_NODES = 1000000
DIM = 32
TOTAL = NUM_NODES * DIM
NUM_CORES = 2
NUM_SUBCORES = 16
NUM_WORKERS = NUM_CORES * NUM_SUBCORES
ELEMS_PER_WORKER = TOTAL // NUM_WORKERS

PIECE = 25000
NPIECES = ELEMS_PER_WORKER // PIECE
NBUF = 4


def _copy_body(w_hbm, out_hbm, b0, b1, b2, b3,
               si0, si1, si2, si3, so0, so1, so2, so3):
    bufs = (b0, b1, b2, b3)
    isems = (si0, si1, si2, si3)
    osems = (so0, so1, so2, so3)
    wid = lax.axis_index("s") * NUM_CORES + lax.axis_index("c")
    base = wid * ELEMS_PER_WORKER

    def in_cp(p, b):
        return pltpu.make_async_copy(
            w_hbm.at[pl.ds(base + p * PIECE, PIECE)], bufs[b], isems[b])

    def out_cp(p, b):
        return pltpu.make_async_copy(
            bufs[b], out_hbm.at[pl.ds(base + p * PIECE, PIECE)], osems[b])

    in_cp(0, 0).start()
    in_cp(1, 1).start()

    def group(i, carry):
        g = i * NBUF
        for b in range(NBUF):
            p = g + b
            nb = (b + 2) % NBUF
            in_cp(p, b).wait()
            out_cp(p, b).start()

            @pl.when(p + 2 >= NBUF)
            def _wait_old_out():
                out_cp(p + 2 - NBUF, nb).wait()

            @pl.when(p + 2 < NPIECES)
            def _start_next_in():
                in_cp(p + 2, nb).start()
        return carry

    lax.fori_loop(0, NPIECES // NBUF, group, 0)

    out_cp(NPIECES - 2, (NPIECES - 2) % NBUF).wait()
    out_cp(NPIECES - 1, (NPIECES - 1) % NBUF).wait()


def kernel(W):
    mesh = plsc.VectorSubcoreMesh(core_axis_name="c", subcore_axis_name="s")
    flat = W.reshape(TOTAL)
    out = pl.kernel(
        _copy_body,
        out_type=jax.ShapeDtypeStruct((TOTAL,), jnp.float32),
        mesh=mesh,
        scratch_types=(
            [pltpu.VMEM((PIECE,), jnp.float32) for _ in range(NBUF)]
            + [pltpu.SemaphoreType.DMA for _ in range(2 * NBUF)]
        ),
    )(flat)
    return out.reshape(NUM_NODES, DIM)
